# baseline (speedup 1.0000x reference)
import functools

import jax
import jax.numpy as jnp
from jax.experimental import pallas as pl
from jax.experimental.pallas import tpu as pltpu

_C_IN = 7
_C_OUT = 4
_K = 5
_CP = 8
_H = 64
_W = 64
_HP = _H + 2 * (_K - 1)
_WP = _W + 2 * (_K - 1)
_HO = _H + _K - 1
_WO = _W + _K - 1
_L_OUT = _HO * _WP
_MAX_SHIFT = (_K - 1) * _WP + (_K - 1)
_L_IN = 5248
_KKC = _K * _K * _CP
_KKC_PAD = 208


def _body(w_ref, x_ref, o_ref, xp_ref, slab_ref):
    xp_ref[...] = jnp.zeros((_CP, _L_IN), jnp.float32)
    for h in range(_H):
        dst = (h + _K - 1) * _WP + (_K - 1)
        xp_ref[0:_C_IN, pl.ds(dst, _W)] = x_ref[0, :, pl.ds(h * _W, _W)]

    for t in range(_K * _K):
        kh, kw = t // _K, t % _K
        shift = kh * _WP + kw
        slab_ref[pl.ds(t * _CP, _CP), :] = xp_ref[:, pl.ds(shift, _L_OUT)]

    row = jax.lax.broadcasted_iota(jnp.int32, (_KKC_PAD - _KKC, _L_OUT), 0)
    slab_ref[pl.ds(_KKC, _KKC_PAD - _KKC), :] = jnp.where(
        row == 0, 1.0, 0.0).astype(jnp.float32)

    v = jnp.dot(w_ref[...], slab_ref[...], preferred_element_type=jnp.float32)

    inner = v * (1.0 + 0.044715 * (v * v)) * 0.7978845608028654
    g = 0.5 * v * (jnp.tanh(inner) + 1.0)
    o_ref[0] = g[:_C_OUT].astype(o_ref.dtype)


def _build_weight_mat(weight, bias):
    w_flip = weight[:, :, ::-1, ::-1]
    w_k = jnp.transpose(w_flip, (2, 3, 0, 1))
    w_k = jnp.pad(w_k, ((0, 0), (0, 0), (0, _CP - _C_IN), (0, 0)))
    w_mat = jnp.transpose(w_k, (3, 0, 1, 2)).reshape(_C_OUT, _KKC)
    w_mat = jnp.concatenate([w_mat, bias.reshape(_C_OUT, 1)], axis=1)
    w_mat = jnp.pad(w_mat, ((0, _CP - _C_OUT), (0, _KKC_PAD - _KKC - 1)))
    return w_mat.astype(jnp.float32)


@jax.jit
def _run(x_nchw, weight, bias):
    n = x_nchw.shape[0]
    x_flat = x_nchw.astype(jnp.float32).reshape(n, _C_IN, _H * _W)
    w_mat = _build_weight_mat(weight, bias)

    out = pl.pallas_call(
        _body,
        out_shape=jax.ShapeDtypeStruct((n, _C_OUT, _L_OUT), jnp.float32),
        grid=(n,),
        in_specs=[
            pl.BlockSpec((_CP, _KKC_PAD), lambda i: (0, 0)),
            pl.BlockSpec((1, _C_IN, _H * _W), lambda i: (i, 0, 0)),
        ],
        out_specs=pl.BlockSpec((1, _C_OUT, _L_OUT), lambda i: (i, 0, 0)),
        scratch_shapes=[
            pltpu.VMEM((_CP, _L_IN), jnp.float32),
            pltpu.VMEM((_KKC_PAD, _L_OUT), jnp.float32),
        ],
        compiler_params=pltpu.CompilerParams(
            dimension_semantics=("parallel",)),
    )(w_mat, x_flat)

    y = out.reshape(n, _C_OUT, _HO, _WP)
    return y[:, :, :, :_WO]


def kernel(x_nchw, weight, bias):
    return _run(x_nchw, weight, bias)


# speedup vs baseline: 1.3766x; 1.3766x over previous
import jax
import jax.numpy as jnp
from jax.experimental import pallas as pl
from jax.experimental.pallas import tpu as pltpu

_C_IN = 7
_C_OUT = 4
_K = 5
_CP = 8
_H = 64
_W = 64
_HP = _H + 2 * (_K - 1)
_WP = _W + 2 * (_K - 1)
_HO = _H + _K - 1
_WO = _W + _K - 1
_L_OUT = _HO * _WP
_MAX_SHIFT = (_K - 1) * _WP + (_K - 1)
_L_IN = 5248
_KKC = _K * _K * _CP
_KKC_PAD = 208
_NB = 4


def _body(w_ref, x_ref, o_ref, xp_ref, slab_ref):
    for nb in range(_NB):
        xp_ref[...] = jnp.zeros((_CP, _L_IN), jnp.bfloat16)
        for h in range(_H):
            dst = (h + _K - 1) * _WP + (_K - 1)
            xp_ref[0:_C_IN, pl.ds(dst, _W)] = x_ref[nb, :, pl.ds(h * _W, _W)]

        col = nb * _L_OUT
        for t in range(_K * _K):
            kh, kw = t // _K, t % _K
            shift = kh * _WP + kw
            slab_ref[pl.ds(t * _CP, _CP), pl.ds(col, _L_OUT)] = (
                xp_ref[:, pl.ds(shift, _L_OUT)])

        row = jax.lax.broadcasted_iota(
            jnp.int32, (_KKC_PAD - _KKC, _L_OUT), 0)
        slab_ref[pl.ds(_KKC, _KKC_PAD - _KKC), pl.ds(col, _L_OUT)] = (
            jnp.where(row == 0, 1.0, 0.0).astype(jnp.bfloat16))

    v = jnp.dot(w_ref[...], slab_ref[...], preferred_element_type=jnp.float32)

    inner = v * (1.0 + 0.044715 * (v * v)) * 0.7978845608028654
    g = 0.5 * v * (jnp.tanh(inner) + 1.0)
    for nb in range(_NB):
        o_ref[nb] = g[:_C_OUT, nb * _L_OUT:(nb + 1) * _L_OUT].astype(o_ref.dtype)


def _build_weight_mat(weight, bias):
    w_flip = weight[:, :, ::-1, ::-1]
    w_k = jnp.transpose(w_flip, (2, 3, 0, 1))
    w_k = jnp.pad(w_k, ((0, 0), (0, 0), (0, _CP - _C_IN), (0, 0)))
    w_mat = jnp.transpose(w_k, (3, 0, 1, 2)).reshape(_C_OUT, _KKC)
    w_mat = jnp.concatenate([w_mat, bias.reshape(_C_OUT, 1)], axis=1)
    w_mat = jnp.pad(w_mat, ((0, _CP - _C_OUT), (0, _KKC_PAD - _KKC - 1)))
    return w_mat.astype(jnp.bfloat16)


@jax.jit
def _run(x_nchw, weight, bias):
    n = x_nchw.shape[0]
    x_flat = x_nchw.astype(jnp.bfloat16).reshape(n, _C_IN, _H * _W)
    w_mat = _build_weight_mat(weight, bias)

    out = pl.pallas_call(
        _body,
        out_shape=jax.ShapeDtypeStruct((n, _C_OUT, _L_OUT), jnp.float32),
        grid=(n // _NB,),
        in_specs=[
            pl.BlockSpec((_CP, _KKC_PAD), lambda i: (0, 0)),
            pl.BlockSpec((_NB, _C_IN, _H * _W), lambda i: (i, 0, 0)),
        ],
        out_specs=pl.BlockSpec((_NB, _C_OUT, _L_OUT), lambda i: (i, 0, 0)),
        scratch_shapes=[
            pltpu.VMEM((_CP, _L_IN), jnp.bfloat16),
            pltpu.VMEM((_KKC_PAD, _NB * _L_OUT), jnp.bfloat16),
        ],
        compiler_params=pltpu.CompilerParams(
            dimension_semantics=("parallel",)),
    )(w_mat, x_flat)

    y = out.reshape(n, _C_OUT, _HO, _WP)
    return y[:, :, :, :_WO]


def kernel(x_nchw, weight, bias):
    return _run(x_nchw, weight, bias)


# speedup vs baseline: 1.8100x; 1.3148x over previous
import jax
import jax.numpy as jnp
from jax.experimental import pallas as pl
from jax.experimental.pallas import tpu as pltpu

_C_IN = 7
_C_OUT = 4
_K = 5
_CP = 8
_H = 64
_W = 64
_HP = _H + 2 * (_K - 1)
_WP = _W + 2 * (_K - 1)
_HO = _H + _K - 1
_WO = _W + _K - 1
_L_OUT = _HO * _WP
_SEG = 4992
_L_IN = 5376
_KR = _K * _CP
_KR_PAD = 48
_NB = 4


def _body(w_ref, x_ref, o_ref, xp_ref, xs_ref, p_ref):
    for nb in range(_NB):
        xp_ref[...] = jnp.zeros((_CP, _L_IN), jnp.bfloat16)
        for h in range(_H):
            dst = (h + _K - 1) * _WP + (_K - 1)
            xp_ref[0:_C_IN, pl.ds(dst, _W)] = x_ref[nb, :, pl.ds(h * _W, _W)]

        col = nb * _SEG
        for kh in range(_K):
            xs_ref[pl.ds(kh * _CP, _CP), pl.ds(col, _SEG)] = (
                xp_ref[:, pl.ds(kh * _WP, _SEG)])

        row = jax.lax.broadcasted_iota(jnp.int32, (_KR_PAD - _KR, _SEG), 0)
        xs_ref[pl.ds(_KR, _KR_PAD - _KR), pl.ds(col, _SEG)] = jnp.where(
            row == 0, 1.0, 0.0).astype(jnp.bfloat16)

    p_ref[...] = jnp.dot(
        w_ref[...], xs_ref[...], preferred_element_type=jnp.float32)

    for nb in range(_NB):
        col = nb * _SEG
        v = p_ref[0:_CP, pl.ds(col, _L_OUT)]
        for kw in range(1, _K):
            v = v + p_ref[pl.ds(kw * _CP, _CP), pl.ds(col + kw, _L_OUT)]

        inner = v * (1.0 + 0.044715 * (v * v)) * 0.7978845608028654
        g = 0.5 * v * (jnp.tanh(inner) + 1.0)
        o_ref[nb] = g[:_C_OUT].astype(o_ref.dtype)


def _build_weight_mat(weight, bias):
    w_flip = weight[:, :, ::-1, ::-1]
    w_flip = jnp.pad(
        w_flip, ((0, _CP - _C_IN), (0, _CP - _C_OUT), (0, 0), (0, 0)))
    arr = jnp.transpose(w_flip, (3, 1, 2, 0))
    w_mat = arr.reshape(_KR, _KR)
    b_col = jnp.zeros((_K, _CP), jnp.float32).at[0, :_C_OUT].set(bias)
    w_mat = jnp.concatenate([w_mat, b_col.reshape(_KR, 1)], axis=1)
    w_mat = jnp.pad(w_mat, ((0, 0), (0, _KR_PAD - _KR - 1)))
    return w_mat.astype(jnp.bfloat16)


@jax.jit
def _run(x_nchw, weight, bias):
    n = x_nchw.shape[0]
    x_flat = x_nchw.astype(jnp.bfloat16).reshape(n, _C_IN, _H * _W)
    w_mat = _build_weight_mat(weight, bias)

    out = pl.pallas_call(
        _body,
        out_shape=jax.ShapeDtypeStruct((n, _C_OUT, _L_OUT), jnp.float32),
        grid=(n // _NB,),
        in_specs=[
            pl.BlockSpec((_KR, _KR_PAD), lambda i: (0, 0)),
            pl.BlockSpec((_NB, _C_IN, _H * _W), lambda i: (i, 0, 0)),
        ],
        out_specs=pl.BlockSpec((_NB, _C_OUT, _L_OUT), lambda i: (i, 0, 0)),
        scratch_shapes=[
            pltpu.VMEM((_CP, _L_IN), jnp.bfloat16),
            pltpu.VMEM((_KR_PAD, _NB * _SEG), jnp.bfloat16),
            pltpu.VMEM((_KR, _NB * _SEG), jnp.float32),
        ],
        compiler_params=pltpu.CompilerParams(
            dimension_semantics=("parallel",)),
    )(w_mat, x_flat)

    y = out.reshape(n, _C_OUT, _HO, _WP)
    return y[:, :, :, :_WO]


def kernel(x_nchw, weight, bias):
    return _run(x_nchw, weight, bias)


# speedup vs baseline: 1.8732x; 1.0349x over previous
import jax
import jax.numpy as jnp
from jax.experimental import pallas as pl
from jax.experimental.pallas import tpu as pltpu

_C_IN = 7
_C_OUT = 4
_K = 5
_CP = 8
_H = 64
_W = 64
_HP = _H + 2 * (_K - 1)
_WP = _W + 2 * (_K - 1)
_HO = _H + _K - 1
_WO = _W + _K - 1
_L_OUT = _HO * _WP
_SEG = 4992
_L_IN = 5376
_KR = _K * _CP
_KR_PAD = 48
_NB = 8


def _body(w_ref, x_ref, o_ref, xp_ref, xs_ref, p_ref):
    for nb in range(_NB):
        xcol = nb * _L_IN
        xp_ref[:, pl.ds(xcol, _L_IN)] = jnp.zeros((_CP, _L_IN), jnp.bfloat16)
        for h in range(_H):
            dst = xcol + (h + _K - 1) * _WP + (_K - 1)
            xp_ref[0:_C_IN, pl.ds(dst, _W)] = x_ref[
                nb, :, pl.ds(h * _W, _W)].astype(jnp.bfloat16)

        col = nb * _SEG
        for kh in range(_K):
            xs_ref[pl.ds(kh * _CP, _CP), pl.ds(col, _SEG)] = (
                xp_ref[:, pl.ds(xcol + kh * _WP, _SEG)])

        row = jax.lax.broadcasted_iota(jnp.int32, (_KR_PAD - _KR, _SEG), 0)
        xs_ref[pl.ds(_KR, _KR_PAD - _KR), pl.ds(col, _SEG)] = jnp.where(
            row == 0, 1.0, 0.0).astype(jnp.bfloat16)

    p_ref[...] = jnp.dot(
        w_ref[...], xs_ref[...], preferred_element_type=jnp.float32)

    for nb in range(_NB):
        col = nb * _SEG
        v = p_ref[0:_CP, pl.ds(col, _L_OUT)]
        for kw in range(1, _K):
            v = v + p_ref[pl.ds(kw * _CP, _CP), pl.ds(col + kw, _L_OUT)]

        inner = v * (1.0 + 0.044715 * (v * v)) * 0.7978845608028654
        g = 0.5 * v * (jnp.tanh(inner) + 1.0)
        o_ref[nb] = g[:_C_OUT].astype(o_ref.dtype)


def _build_weight_mat(weight, bias):
    w_flip = weight[:, :, ::-1, ::-1]
    w_flip = jnp.pad(
        w_flip, ((0, _CP - _C_IN), (0, _CP - _C_OUT), (0, 0), (0, 0)))
    arr = jnp.transpose(w_flip, (3, 1, 2, 0))
    w_mat = arr.reshape(_KR, _KR)
    b_col = jnp.zeros((_K, _CP), jnp.float32).at[0, :_C_OUT].set(bias)
    w_mat = jnp.concatenate([w_mat, b_col.reshape(_KR, 1)], axis=1)
    w_mat = jnp.pad(w_mat, ((0, 0), (0, _KR_PAD - _KR - 1)))
    return w_mat.astype(jnp.bfloat16)


@jax.jit
def _run(x_nchw, weight, bias):
    n = x_nchw.shape[0]
    x_flat = x_nchw.reshape(n, _C_IN, _H * _W)
    w_mat = _build_weight_mat(weight, bias)

    out = pl.pallas_call(
        _body,
        out_shape=jax.ShapeDtypeStruct((n, _C_OUT, _L_OUT), jnp.float32),
        grid=(n // _NB,),
        in_specs=[
            pl.BlockSpec((_KR, _KR_PAD), lambda i: (0, 0)),
            pl.BlockSpec((_NB, _C_IN, _H * _W), lambda i: (i, 0, 0)),
        ],
        out_specs=pl.BlockSpec((_NB, _C_OUT, _L_OUT), lambda i: (i, 0, 0)),
        scratch_shapes=[
            pltpu.VMEM((_CP, _NB * _L_IN), jnp.bfloat16),
            pltpu.VMEM((_KR_PAD, _NB * _SEG), jnp.bfloat16),
            pltpu.VMEM((_KR, _NB * _SEG), jnp.float32),
        ],
        compiler_params=pltpu.CompilerParams(
            dimension_semantics=("parallel",)),
    )(w_mat, x_flat)

    y = out.reshape(n, _C_OUT, _HO, _WP)
    return y[:, :, :, :_WO]


def kernel(x_nchw, weight, bias):
    return _run(x_nchw, weight, bias)
